# baseline (speedup 1.0000x reference)
import jax
import jax.numpy as jnp
from jax.experimental import pallas as pl
from jax.experimental.pallas import tpu as pltpu


def _ctg_body(x_ref, w_ref, b_ref, m_ref, o_ref):
    _, c_in, j, _ = x_ref.shape
    c_out = o_ref.shape[1]
    x2 = x_ref[0].reshape(c_in * j, 128).astype(jnp.bfloat16)
    acc = jax.lax.dot_general(
        w_ref[...], x2,
        dimension_numbers=(((1,), (0,)), ((), ())),
        preferred_element_type=jnp.float32)
    acc3 = acc.reshape(c_out, j, 128)
    o_ref[0] = (acc3 + b_ref[...]) * m_ref[...]


def kernel(x, A, weight, bias, mask, *, j_tile=8):
    N, C_in, T, V = x.shape
    C_out = weight.shape[0]
    TV = T * V
    assert TV % 128 == 0, "spatial extent must be a multiple of 128"
    S = TV // 128
    if S % j_tile != 0:
        j_tile = S
    J = j_tile
    grid = (N, S // J)

    x4 = x.reshape(N, C_in, S, 128)
    m4 = mask.reshape(N, S, 128)

    w2 = weight.reshape(C_out, C_in).astype(jnp.bfloat16)
    w_bd = jnp.kron(w2, jnp.eye(J, dtype=jnp.bfloat16))
    b3 = bias.reshape(C_out, 1, 1).astype(jnp.float32)

    out4 = pl.pallas_call(
        _ctg_body,
        out_shape=jax.ShapeDtypeStruct((N, C_out, S, 128), x.dtype),
        grid=grid,
        in_specs=[
            pl.BlockSpec((1, C_in, J, 128), lambda n, s: (n, 0, s, 0)),
            pl.BlockSpec((C_out * J, C_in * J), lambda n, s: (0, 0)),
            pl.BlockSpec((C_out, 1, 1), lambda n, s: (0, 0, 0)),
            pl.BlockSpec((1, J, 128), lambda n, s: (n, s, 0)),
        ],
        out_specs=pl.BlockSpec((1, C_out, J, 128), lambda n, s: (n, 0, s, 0)),
        compiler_params=pltpu.CompilerParams(
            dimension_semantics=("parallel", "parallel")),
        cost_estimate=pl.CostEstimate(
            flops=2 * N * C_out * C_in * TV,
            transcendentals=0,
            bytes_accessed=4 * (N * C_in * TV + N * C_out * TV + N * TV)),
    )(x4, w_bd, b3, m4)

    return out4.reshape(N, C_out, T, V), A


# speedup vs baseline: 1.3122x; 1.3122x over previous
import jax
import jax.numpy as jnp
from jax.experimental import pallas as pl
from jax.experimental.pallas import tpu as pltpu


def _ctg_body(x_ref, w_ref, b_ref, m_ref, o_ref):
    bn = x_ref.shape[0]
    for i in range(bn):
        xb = x_ref[i].astype(jnp.bfloat16)
        acc = jax.lax.dot_general(
            w_ref[...], xb,
            dimension_numbers=(((1,), (0,)), ((), ())),
            preferred_element_type=jnp.float32)
        o_ref[i] = (acc + b_ref[...]) * m_ref[i]


def kernel(x, A, weight, bias, mask, *, bn=1, tv_tile=4096):
    N, C_in, T, V = x.shape
    C_out = weight.shape[0]
    TV = T * V
    if TV % tv_tile != 0:
        tv_tile = TV
    if N % bn != 0:
        bn = 1
    grid = (N // bn, TV // tv_tile)

    x3 = x.reshape(N, C_in, TV)
    w2 = weight.reshape(C_out, C_in).astype(jnp.bfloat16)
    b2 = bias.reshape(C_out, 1).astype(jnp.float32)
    m3 = mask.reshape(N, 1, TV).astype(x.dtype)

    out3 = pl.pallas_call(
        _ctg_body,
        out_shape=jax.ShapeDtypeStruct((N, C_out, TV), x.dtype),
        grid=grid,
        in_specs=[
            pl.BlockSpec((bn, C_in, tv_tile), lambda n, j: (n, 0, j)),
            pl.BlockSpec((C_out, C_in), lambda n, j: (0, 0)),
            pl.BlockSpec((C_out, 1), lambda n, j: (0, 0)),
            pl.BlockSpec((bn, 1, tv_tile), lambda n, j: (n, 0, j)),
        ],
        out_specs=pl.BlockSpec((bn, C_out, tv_tile), lambda n, j: (n, 0, j)),
        compiler_params=pltpu.CompilerParams(
            dimension_semantics=("parallel", "parallel")),
        cost_estimate=pl.CostEstimate(
            flops=2 * N * C_out * C_in * TV,
            transcendentals=0,
            bytes_accessed=4 * (N * C_in * TV + N * C_out * TV + N * TV)),
    )(x3, w2, b2, m3)

    return out3.reshape(N, C_out, T, V), A


# speedup vs baseline: 1.3703x; 1.0443x over previous
import jax
import jax.numpy as jnp
from jax.experimental import pallas as pl
from jax.experimental.pallas import tpu as pltpu


def _ctg_body(x_ref, w_ref, b_ref, m_ref, o_ref):
    bn = x_ref.shape[0]
    for i in range(bn):
        xb = x_ref[i].astype(jnp.bfloat16)
        acc = jax.lax.dot_general(
            w_ref[...], xb,
            dimension_numbers=(((1,), (0,)), ((), ())),
            preferred_element_type=jnp.float32)
        o_ref[i] = (acc + b_ref[...]) * m_ref[i]


def kernel(x, A, weight, bias, mask, *, bn=2, tv_tile=4096):
    N, C_in, T, V = x.shape
    C_out = weight.shape[0]
    TV = T * V
    if TV % tv_tile != 0:
        tv_tile = TV
    if N % bn != 0:
        bn = 1
    grid = (N // bn, TV // tv_tile)

    x3 = x.reshape(N, C_in, TV)
    w2 = weight.reshape(C_out, C_in).astype(jnp.bfloat16)
    b2 = bias.reshape(C_out, 1).astype(jnp.float32)
    m3 = mask.reshape(N, 1, TV).astype(x.dtype)

    out3 = pl.pallas_call(
        _ctg_body,
        out_shape=jax.ShapeDtypeStruct((N, C_out, TV), x.dtype),
        grid=grid,
        in_specs=[
            pl.BlockSpec((bn, C_in, tv_tile), lambda n, j: (n, 0, j)),
            pl.BlockSpec((C_out, C_in), lambda n, j: (0, 0)),
            pl.BlockSpec((C_out, 1), lambda n, j: (0, 0)),
            pl.BlockSpec((bn, 1, tv_tile), lambda n, j: (n, 0, j)),
        ],
        out_specs=pl.BlockSpec((bn, C_out, tv_tile), lambda n, j: (n, 0, j)),
        compiler_params=pltpu.CompilerParams(
            dimension_semantics=("parallel", "parallel")),
        cost_estimate=pl.CostEstimate(
            flops=2 * N * C_out * C_in * TV,
            transcendentals=0,
            bytes_accessed=4 * (N * C_in * TV + N * C_out * TV + N * TV)),
    )(x3, w2, b2, m3)

    return out3.reshape(N, C_out, T, V), A


# speedup vs baseline: 1.3834x; 1.0095x over previous
import jax
import jax.numpy as jnp
from jax.experimental import pallas as pl
from jax.experimental.pallas import tpu as pltpu


def _ctg_body(x_ref, w_ref, b_ref, m_ref, o_ref):
    bn = x_ref.shape[0]
    for i in range(bn):
        xb = x_ref[i].astype(jnp.bfloat16)
        acc = jax.lax.dot_general(
            w_ref[...], xb,
            dimension_numbers=(((1,), (0,)), ((), ())),
            preferred_element_type=jnp.float32)
        o_ref[i] = (acc + b_ref[...]) * m_ref[i]


def kernel(x, A, weight, bias, mask, *, bn=4, tv_tile=4096):
    N, C_in, T, V = x.shape
    C_out = weight.shape[0]
    TV = T * V
    if TV % tv_tile != 0:
        tv_tile = TV
    if N % bn != 0:
        bn = 1
    grid = (N // bn, TV // tv_tile)

    x3 = x.reshape(N, C_in, TV)
    w2 = weight.reshape(C_out, C_in).astype(jnp.bfloat16)
    b2 = bias.reshape(C_out, 1).astype(jnp.float32)
    m3 = mask.reshape(N, 1, TV).astype(x.dtype)

    out3 = pl.pallas_call(
        _ctg_body,
        out_shape=jax.ShapeDtypeStruct((N, C_out, TV), x.dtype),
        grid=grid,
        in_specs=[
            pl.BlockSpec((bn, C_in, tv_tile), lambda n, j: (n, 0, j)),
            pl.BlockSpec((C_out, C_in), lambda n, j: (0, 0)),
            pl.BlockSpec((C_out, 1), lambda n, j: (0, 0)),
            pl.BlockSpec((bn, 1, tv_tile), lambda n, j: (n, 0, j)),
        ],
        out_specs=pl.BlockSpec((bn, C_out, tv_tile), lambda n, j: (n, 0, j)),
        compiler_params=pltpu.CompilerParams(
            dimension_semantics=("parallel", "parallel")),
        cost_estimate=pl.CostEstimate(
            flops=2 * N * C_out * C_in * TV,
            transcendentals=0,
            bytes_accessed=4 * (N * C_in * TV + N * C_out * TV + N * TV)),
    )(x3, w2, b2, m3)

    return out3.reshape(N, C_out, T, V), A
